# Initial kernel scaffold; baseline (speedup 1.0000x reference)
import jax
import jax.numpy as jnp
from jax.experimental import pallas as pl


def kernel(x, y):
    raise NotImplementedError("write your pallas kernel here")


# speedup vs baseline: 1.0083x; 1.0083x over previous
import functools

import jax
import jax.numpy as jnp
from jax import lax
from jax.experimental import pallas as pl
from jax.experimental.pallas import tpu as pltpu
from jax.experimental.pallas import tpu_sc as plsc

B, C, H, W = 4, 96, 384, 384
N, S = 4096, 16
Q = B * N * S
NC, NS, L = 2, 16, 16
NW = NC * NS
QPW = Q // NW
P = 128
NCHUNK = QPW // P
CL = C // L


def _sc_body(xt, gx, gy, out,
             gx_v, gy_v,
             w00_v, w01_v, w10_v, w11_v,
             i00_v, i01_v, i10_v, i11_v,
             r00, r01, r10, r11, out_v, sem):
    cid = lax.axis_index("c")
    sid = lax.axis_index("s")
    wid = sid * NC + cid
    qbase = wid * QPW
    base = (qbase // (N * S)) * (H * W)

    pltpu.sync_copy(gx.at[pl.ds(qbase, QPW)], gx_v)
    pltpu.sync_copy(gy.at[pl.ds(qbase, QPW)], gy_v)

    def chunk(k, _):
        off = k * P

        def grp(j, _):
            sl = pl.ds(off + j * L, L)
            dst = pl.ds(j * L, L)
            px = (gx_v[sl] + 1.0) * (0.5 * (W - 1))
            py = (gy_v[sl] + 1.0) * (0.5 * (H - 1))
            px = jnp.minimum(jnp.maximum(px, 0.0), float(W - 1))
            py = jnp.minimum(jnp.maximum(py, 0.0), float(H - 1))
            x0 = px.astype(jnp.int32)
            y0 = py.astype(jnp.int32)
            fx = px - x0.astype(jnp.float32)
            fy = py - y0.astype(jnp.float32)
            dx = jnp.minimum(x0 + 1, W - 1) - x0
            dy = jnp.minimum(y0 + 1, H - 1) - y0
            i00 = base + y0 * W + x0
            i10 = i00 + dy * W
            gx1 = 1.0 - fx
            gy1 = 1.0 - fy
            w00_v[dst] = gx1 * gy1
            w01_v[dst] = fx * gy1
            w10_v[dst] = gx1 * fy
            w11_v[dst] = fx * fy
            i00_v[dst] = i00
            i01_v[dst] = i00 + dx
            i10_v[dst] = i10
            i11_v[dst] = i10 + dx
            return 0

        lax.fori_loop(0, P // L, grp, 0)

        cp0 = pltpu.async_copy(xt.at[i00_v], r00, sem)
        cp1 = pltpu.async_copy(xt.at[i01_v], r01, sem)
        cp2 = pltpu.async_copy(xt.at[i10_v], r10, sem)
        cp3 = pltpu.async_copy(xt.at[i11_v], r11, sem)
        cp0.wait()
        cp1.wait()
        cp2.wait()
        cp3.wait()

        def blend(g, _):
            gsl = pl.ds(g * L, L)
            w00 = w00_v[gsl]
            w01 = w01_v[gsl]
            w10 = w10_v[gsl]
            w11 = w11_v[gsl]
            for ii in range(L):
                i = g * L + ii
                a = w00[ii]
                b = w01[ii]
                c = w10[ii]
                d = w11[ii]
                for cc in range(CL):
                    sl = pl.ds(cc * L, L)
                    out_v[i, sl] = (a * r00[i, sl] + b * r01[i, sl]
                                    + c * r10[i, sl] + d * r11[i, sl])
            return 0

        lax.fori_loop(0, P // L, blend, 0)

        pltpu.sync_copy(out_v, out.at[pl.ds(qbase + off, P)])
        return 0

    lax.fori_loop(0, NCHUNK, chunk, 0)


@jax.jit
def _sc_call(xt, gx, gy):
    mesh = plsc.VectorSubcoreMesh(core_axis_name="c", subcore_axis_name="s")
    return pl.kernel(
        _sc_body,
        out_type=jax.ShapeDtypeStruct((Q, C), jnp.float32),
        mesh=mesh,
        compiler_params=pltpu.CompilerParams(use_tc_tiling_on_sc=False),
        scratch_types=[
            pltpu.VMEM((QPW,), jnp.float32),
            pltpu.VMEM((QPW,), jnp.float32),
            pltpu.VMEM((P,), jnp.float32),
            pltpu.VMEM((P,), jnp.float32),
            pltpu.VMEM((P,), jnp.float32),
            pltpu.VMEM((P,), jnp.float32),
            pltpu.VMEM((P,), jnp.int32),
            pltpu.VMEM((P,), jnp.int32),
            pltpu.VMEM((P,), jnp.int32),
            pltpu.VMEM((P,), jnp.int32),
            pltpu.VMEM((P, C), jnp.float32),
            pltpu.VMEM((P, C), jnp.float32),
            pltpu.VMEM((P, C), jnp.float32),
            pltpu.VMEM((P, C), jnp.float32),
            pltpu.VMEM((P, C), jnp.float32),
            pltpu.SemaphoreType.DMA,
        ],
    )(xt, gx, gy)


def kernel(x, y):
    xt = jnp.transpose(x, (0, 2, 3, 1)).reshape(B * H * W, C)
    gx = y[..., 0].reshape(Q)
    gy = y[..., 1].reshape(Q)
    out = _sc_call(xt, gx, gy)
    return out.reshape(B, N, S, C)


# speedup vs baseline: 1.2185x; 1.2085x over previous
import functools

import jax
import jax.numpy as jnp
from jax import lax
from jax.experimental import pallas as pl
from jax.experimental.pallas import tpu as pltpu
from jax.experimental.pallas import tpu_sc as plsc

B, C, H, W = 4, 96, 384, 384
CP = 128
N, S = 4096, 16
Q = B * N * S
NC, NS, L = 2, 16, 16
NW = NC * NS
QPW = Q // NW
P = 128
NCHUNK = QPW // P
CL = C // L


def _sc_body(xt, gx, gy, out,
             gx_v, gy_v,
             w00_v, w01_v, w10_v, w11_v,
             i00_v, i01_v, i10_v, i11_v,
             r00, r01, r10, r11, out_v, sem):
    cid = lax.axis_index("c")
    sid = lax.axis_index("s")
    wid = sid * NC + cid
    qbase = wid * QPW
    base = (qbase // (N * S)) * (H * W)

    pltpu.sync_copy(gx.at[pl.ds(qbase, QPW)], gx_v)
    pltpu.sync_copy(gy.at[pl.ds(qbase, QPW)], gy_v)

    def chunk(k, _):
        off = k * P

        def grp(j, _):
            sl = pl.ds(off + j * L, L)
            dst = pl.ds(j * L, L)
            px = (gx_v[sl] + 1.0) * (0.5 * (W - 1))
            py = (gy_v[sl] + 1.0) * (0.5 * (H - 1))
            px = jnp.minimum(jnp.maximum(px, 0.0), float(W - 1))
            py = jnp.minimum(jnp.maximum(py, 0.0), float(H - 1))
            x0 = px.astype(jnp.int32)
            y0 = py.astype(jnp.int32)
            fx = px - x0.astype(jnp.float32)
            fy = py - y0.astype(jnp.float32)
            dx = jnp.minimum(x0 + 1, W - 1) - x0
            dy = jnp.minimum(y0 + 1, H - 1) - y0
            i00 = base + y0 * W + x0
            i10 = i00 + dy * W
            gx1 = 1.0 - fx
            gy1 = 1.0 - fy
            w00_v[dst] = gx1 * gy1
            w01_v[dst] = fx * gy1
            w10_v[dst] = gx1 * fy
            w11_v[dst] = fx * fy
            i00_v[dst] = i00
            i01_v[dst] = i00 + dx
            i10_v[dst] = i10
            i11_v[dst] = i10 + dx
            return 0

        lax.fori_loop(0, P // L, grp, 0)

        cp0 = pltpu.async_copy(xt.at[i00_v], r00, sem)
        cp1 = pltpu.async_copy(xt.at[i01_v], r01, sem)
        cp2 = pltpu.async_copy(xt.at[i10_v], r10, sem)
        cp3 = pltpu.async_copy(xt.at[i11_v], r11, sem)
        cp0.wait()
        cp1.wait()
        cp2.wait()
        cp3.wait()

        def blend(g, _):
            gsl = pl.ds(g * L, L)
            w00 = w00_v[gsl]
            w01 = w01_v[gsl]
            w10 = w10_v[gsl]
            w11 = w11_v[gsl]
            for ii in range(L):
                i = g * L + ii
                a = w00[ii]
                b = w01[ii]
                c = w10[ii]
                d = w11[ii]
                for cc in range(CL):
                    sl = pl.ds(cc * L, L)
                    out_v[i, sl] = (a * r00[i, sl] + b * r01[i, sl]
                                    + c * r10[i, sl] + d * r11[i, sl])
            return 0

        lax.fori_loop(0, P // L, blend, 0)

        pltpu.sync_copy(out_v, out.at[pl.ds(qbase + off, P)])
        return 0

    lax.fori_loop(0, NCHUNK, chunk, 0)


@jax.jit
def _sc_call(xt, gx, gy):
    mesh = plsc.VectorSubcoreMesh(core_axis_name="c", subcore_axis_name="s")
    return pl.kernel(
        _sc_body,
        out_type=jax.ShapeDtypeStruct((Q, CP), jnp.float32),
        mesh=mesh,
        compiler_params=pltpu.CompilerParams(use_tc_tiling_on_sc=True),
        scratch_types=[
            pltpu.VMEM((QPW,), jnp.float32),
            pltpu.VMEM((QPW,), jnp.float32),
            pltpu.VMEM((P,), jnp.float32),
            pltpu.VMEM((P,), jnp.float32),
            pltpu.VMEM((P,), jnp.float32),
            pltpu.VMEM((P,), jnp.float32),
            pltpu.VMEM((P,), jnp.int32),
            pltpu.VMEM((P,), jnp.int32),
            pltpu.VMEM((P,), jnp.int32),
            pltpu.VMEM((P,), jnp.int32),
            pltpu.VMEM((P, CP), jnp.float32),
            pltpu.VMEM((P, CP), jnp.float32),
            pltpu.VMEM((P, CP), jnp.float32),
            pltpu.VMEM((P, CP), jnp.float32),
            pltpu.VMEM((P, CP), jnp.float32),
            pltpu.SemaphoreType.DMA,
        ],
    )(xt, gx, gy)


def kernel(x, y):
    xp = jnp.pad(x, ((0, 0), (0, CP - C), (0, 0), (0, 0)))
    xt = jnp.transpose(xp, (0, 2, 3, 1)).reshape(B * H * W, CP)
    gx = y[..., 0].reshape(Q)
    gy = y[..., 1].reshape(Q)
    out = _sc_call(xt, gx, gy)
    return out.reshape(B, N, S, CP)[..., :C]
